# baseline (speedup 1.0000x reference)
import math

import jax
import jax.numpy as jnp
from jax.experimental import pallas as pl
from jax.experimental.pallas import tpu as pltpu

_G = 8
_N_REAL = 800
_N_ALL = 1000
_HW = 128 * 128
_LN2 = math.log(2.0)


def _bce_body(idx_ref, np_ref, *refs):
    p_refs = refs[:_G]
    m_refs = refs[_G:2 * _G]
    s_ref = refs[2 * _G]
    o_ref = refs[2 * _G + 1]
    acc_ref = refs[2 * _G + 2]
    i = pl.program_id(0)

    @pl.when(i == 0)
    def _():
        acc_ref[0] = 0.0

    npos = np_ref[0]
    for j in range(_G):
        @pl.when(i * _G + j < npos)
        def _(j=j):
            x = p_refs[j][...]
            m = m_refs[j][...]
            z = (m >= 0.5).astype(jnp.float32)
            bce = jnp.maximum(x, 0.0) - x * z + jnp.log1p(jnp.exp(-jnp.abs(x)))
            acc_ref[0] += jnp.sum(bce)

    @pl.when(i == pl.num_programs(0) - 1)
    def _():
        s = s_ref[...]
        posf = (s > 0.0).astype(jnp.float32)
        flat = (jax.lax.broadcasted_iota(jnp.int32, (8, 128), 0) * 128
                + jax.lax.broadcasted_iota(jnp.int32, (8, 128), 1))
        denom = jnp.sum(posf)
        pad_cnt = jnp.sum(jnp.where(flat >= _N_REAL, posf, 0.0))
        loss = (acc_ref[0] + pad_cnt * (_HW * _LN2)) / denom
        o_ref[...] = jnp.reshape(loss, (1, 1))


def _gather_imap(j):
    def f(i, idx_ref, np_ref):
        k = jnp.minimum(i * _G + j, jnp.maximum(np_ref[0] - 1, 0))
        return (idx_ref[k], 0, 0)
    return f


def kernel(mask_preds, masks, scores):
    preds3 = mask_preds[0]
    masks3 = masks[0, :_N_REAL]
    scores_f = scores.reshape(-1)

    pos800 = scores_f[:_N_REAL] > 0.0
    idx = jnp.nonzero(pos800, size=_N_REAL, fill_value=0)[0].astype(jnp.int32)
    npos = jnp.sum(pos800).astype(jnp.int32).reshape(1)
    s_pad = jnp.pad(scores_f, (0, 1024 - _N_ALL),
                    constant_values=-1.0).reshape(8, 128)

    grid = _N_REAL // _G
    inst_spec = [pl.BlockSpec((1, 128, 128), _gather_imap(j)) for j in range(_G)]
    grid_spec = pltpu.PrefetchScalarGridSpec(
        num_scalar_prefetch=2,
        grid=(grid,),
        in_specs=inst_spec + inst_spec
        + [pl.BlockSpec((8, 128), lambda i, *_: (0, 0))],
        out_specs=pl.BlockSpec((1, 1), lambda i, *_: (0, 0)),
        scratch_shapes=[pltpu.SMEM((1,), jnp.float32)],
    )
    out = pl.pallas_call(
        _bce_body,
        grid_spec=grid_spec,
        out_shape=jax.ShapeDtypeStruct((1, 1), jnp.float32),
        compiler_params=pltpu.CompilerParams(
            dimension_semantics=("arbitrary",)),
    )(idx, npos, *([preds3] * _G), *([masks3] * _G), s_pad)
    return out[0, 0]


# speedup vs baseline: 1.3757x; 1.3757x over previous
import math

import jax
import jax.numpy as jnp
from jax.experimental import pallas as pl
from jax.experimental.pallas import tpu as pltpu

_G = 8
_N_REAL = 800
_N_ALL = 1000
_HW = 128 * 128
_LN2 = math.log(2.0)


_LOG2E = 1.0 / _LN2
_NEG = -1e30


def _bce_body(idx_ref, np_ref, *refs):
    p_refs = refs[:_G]
    m_refs = refs[_G:2 * _G]
    s_ref = refs[2 * _G]
    o_ref = refs[2 * _G + 1]
    acc_t = refs[2 * _G + 2]
    acc_x = refs[2 * _G + 3]
    i = pl.program_id(0)
    npos = np_ref[0]

    @pl.when(i == 0)
    def _():
        acc_t[...] = jnp.zeros((128, 128), jnp.float32)
        acc_x[...] = jnp.zeros((128, 128), jnp.float32)

    @pl.when(i * _G < npos)
    def _():
        t_tot = acc_t[...]
        x_tot = acc_x[...]
        for j in range(_G):
            x = p_refs[j][0]
            m = m_refs[j][0]
            valid = (i * _G + j) < npos
            xc = jnp.where(valid, x, _NEG)
            t_tot = t_tot + jnp.log2(1.0 + jnp.exp2(xc * _LOG2E))
            xz = jnp.where(m >= 0.5, x, 0.0)
            x_tot = x_tot + jnp.where(valid, xz, 0.0)
        acc_t[...] = t_tot
        acc_x[...] = x_tot

    @pl.when(i == pl.num_programs(0) - 1)
    def _():
        s = s_ref[...]
        posf = (s > 0.0).astype(jnp.float32)
        flat = (jax.lax.broadcasted_iota(jnp.int32, (8, 128), 0) * 128
                + jax.lax.broadcasted_iota(jnp.int32, (8, 128), 1))
        denom = jnp.sum(posf)
        pad_cnt = jnp.sum(jnp.where(flat >= _N_REAL, posf, 0.0))
        loss_sum = _LN2 * jnp.sum(acc_t[...]) - jnp.sum(acc_x[...])
        loss = (loss_sum + pad_cnt * (_HW * _LN2)) / denom
        o_ref[...] = jnp.reshape(loss, (1, 1))


def _gather_imap(j):
    def f(i, idx_ref, np_ref):
        k = jnp.minimum(i * _G + j, jnp.maximum(np_ref[0] - 1, 0))
        return (idx_ref[k], 0, 0)
    return f


def kernel(mask_preds, masks, scores):
    preds3 = mask_preds[0]
    masks3 = masks[0, :_N_REAL]
    scores_f = scores.reshape(-1)

    pos800 = scores_f[:_N_REAL] > 0.0
    idx = jnp.nonzero(pos800, size=_N_REAL, fill_value=0)[0].astype(jnp.int32)
    npos = jnp.sum(pos800).astype(jnp.int32).reshape(1)
    s_pad = jnp.pad(scores_f, (0, 1024 - _N_ALL),
                    constant_values=-1.0).reshape(8, 128)

    grid = _N_REAL // _G
    inst_spec = [pl.BlockSpec((1, 128, 128), _gather_imap(j)) for j in range(_G)]
    grid_spec = pltpu.PrefetchScalarGridSpec(
        num_scalar_prefetch=2,
        grid=(grid,),
        in_specs=inst_spec + inst_spec
        + [pl.BlockSpec((8, 128), lambda i, *_: (0, 0))],
        out_specs=pl.BlockSpec((1, 1), lambda i, *_: (0, 0)),
        scratch_shapes=[pltpu.VMEM((128, 128), jnp.float32),
                        pltpu.VMEM((128, 128), jnp.float32)],
    )
    out = pl.pallas_call(
        _bce_body,
        grid_spec=grid_spec,
        out_shape=jax.ShapeDtypeStruct((1, 1), jnp.float32),
        compiler_params=pltpu.CompilerParams(
            dimension_semantics=("arbitrary",)),
    )(idx, npos, *([preds3] * _G), *([masks3] * _G), s_pad)
    return out[0, 0]
